# baseline (speedup 1.0000x reference)
import functools

import jax
import jax.numpy as jnp
from jax import lax
from jax.experimental import pallas as pl
from jax.experimental.pallas import tpu as pltpu
from jax.experimental.pallas import tpu_sc as plsc

_POS_OFFSET = 2


def _sc_broadcast(weights, nb, t):
    d = weights.shape[1]
    NC, NS = 2, 16
    NW = NC * NS
    rows_per_w = t // NW
    CHUNK = 64
    n_chunks = rows_per_w // CHUNK
    L = 16
    mesh = plsc.VectorSubcoreMesh(core_axis_name="c", subcore_axis_name="s")

    @functools.partial(
        pl.kernel,
        mesh=mesh,
        out_type=jax.ShapeDtypeStruct((nb, t, d), weights.dtype),
        scratch_types=[
            pltpu.VMEM((CHUNK, d), weights.dtype),
            pltpu.VMEM((CHUNK, d), weights.dtype),
            [pltpu.VMEM((CHUNK,), jnp.int32) for _ in range(n_chunks)],
            pltpu.SemaphoreType.DMA,
            pltpu.SemaphoreType.DMA,
            pltpu.SemaphoreType.DMA,
            pltpu.SemaphoreType.DMA,
        ],
    )
    def _posemb(w_hbm, out_hbm, buf_a, buf_b, idxs, sin_a, sin_b, sout_a, sout_b):
        wid = lax.axis_index("s") * NC + lax.axis_index("c")
        base = wid * rows_per_w
        bufs, sins, souts = [buf_a, buf_b], [sin_a, sin_b], [sout_a, sout_b]

        lane = lax.iota(jnp.int32, L)
        for ci in range(n_chunks):
            for j in range(CHUNK // L):
                idxs[ci][pl.ds(j * L, L)] = (
                    _POS_OFFSET + base + ci * CHUNK + j * L) + lane

        def start_in(ci):
            return pltpu.async_copy(
                w_hbm.at[idxs[ci]], bufs[ci % 2], sins[ci % 2])

        def start_outs(ci):
            r0 = base + ci * CHUNK
            return [
                pltpu.async_copy(
                    bufs[ci % 2],
                    out_hbm.at[bi, pl.ds(r0, CHUNK)],
                    souts[ci % 2])
                for bi in range(nb)
            ]

        ins, outs = {}, {}
        for ci in range(min(2, n_chunks)):
            ins[ci] = start_in(ci)
        for ci in range(min(2, n_chunks)):
            ins[ci].wait()
            outs[ci] = start_outs(ci)
        for ci in range(2, n_chunks):
            for cp in outs[ci - 2]:
                cp.wait()
            ins[ci] = start_in(ci)
            ins[ci].wait()
            outs[ci] = start_outs(ci)
        for ci in range(max(0, n_chunks - 2), n_chunks):
            for cp in outs[ci]:
                cp.wait()

    return _posemb(weights)


def kernel(input, weights):
    b, t = input.shape
    half = _sc_broadcast(weights, b // 2, t)
    half2 = _sc_broadcast(weights, b - b // 2, t)
    return jnp.concatenate([half, half2], axis=0)


# speedup vs baseline: 1.7435x; 1.7435x over previous
import functools

import jax
import jax.numpy as jnp
from jax import lax
from jax.experimental import pallas as pl
from jax.experimental.pallas import tpu as pltpu
from jax.experimental.pallas import tpu_sc as plsc

_POS_OFFSET = 2


def kernel(input, weights):
    b, t = input.shape
    d = weights.shape[1]

    NC, NS = 2, 16
    NW = NC * NS
    rows_per_w = t // NW
    CHUNK = 32
    NBUF = 4
    n_chunks = rows_per_w // CHUNK
    L = 16
    mesh = plsc.VectorSubcoreMesh(core_axis_name="c", subcore_axis_name="s")

    @functools.partial(
        pl.kernel,
        mesh=mesh,
        out_type=jax.ShapeDtypeStruct((b, t, d), weights.dtype),
        scratch_types=[
            [pltpu.VMEM((CHUNK, d), weights.dtype) for _ in range(NBUF)],
            [pltpu.VMEM((CHUNK,), jnp.int32) for _ in range(n_chunks)],
            [pltpu.SemaphoreType.DMA for _ in range(NBUF)],
            [pltpu.SemaphoreType.DMA for _ in range(NBUF)],
        ],
    )
    def _posemb(w_hbm, out_hbm, bufs, idxs, sins, souts):
        wid = lax.axis_index("s") * NC + lax.axis_index("c")
        base = wid * rows_per_w

        lane = lax.iota(jnp.int32, L)
        for ci in range(n_chunks):
            for j in range(CHUNK // L):
                idxs[ci][pl.ds(j * L, L)] = (
                    _POS_OFFSET + base + ci * CHUNK + j * L) + lane

        def start_in(ci):
            return pltpu.async_copy(
                w_hbm.at[idxs[ci]], bufs[ci % NBUF], sins[ci % NBUF])

        def start_outs(ci):
            r0 = base + ci * CHUNK
            return [
                pltpu.async_copy(
                    bufs[ci % NBUF],
                    out_hbm.at[bi, pl.ds(r0, CHUNK)],
                    souts[ci % NBUF])
                for bi in range(b)
            ]

        ins, outs = {}, {}
        for ci in range(min(NBUF, n_chunks)):
            ins[ci] = start_in(ci)
        for ci in range(min(NBUF, n_chunks)):
            ins[ci].wait()
            outs[ci] = start_outs(ci)
        for ci in range(NBUF, n_chunks):
            for cp in outs[ci - NBUF]:
                cp.wait()
            ins[ci] = start_in(ci)
            ins[ci].wait()
            outs[ci] = start_outs(ci)
        for ci in range(max(0, n_chunks - NBUF), n_chunks):
            for cp in outs[ci]:
                cp.wait()

    return _posemb(weights)


# speedup vs baseline: 1.7751x; 1.0181x over previous
import functools

import jax
import jax.numpy as jnp
from jax import lax
from jax.experimental import pallas as pl
from jax.experimental.pallas import tpu as pltpu
from jax.experimental.pallas import tpu_sc as plsc

_POS_OFFSET = 2


def kernel(input, weights):
    b, t = input.shape
    d = weights.shape[1]

    NC, NS = 2, 16
    NW = NC * NS
    rows_per_w = t // NW
    CHUNK = 64
    NBUF = 2
    n_chunks = rows_per_w // CHUNK
    L = 16
    mesh = plsc.VectorSubcoreMesh(core_axis_name="c", subcore_axis_name="s")

    @functools.partial(
        pl.kernel,
        mesh=mesh,
        out_type=jax.ShapeDtypeStruct((b, t, d), weights.dtype),
        scratch_types=[
            [pltpu.VMEM((CHUNK, d), weights.dtype) for _ in range(NBUF)],
            [pltpu.VMEM((CHUNK,), jnp.int32) for _ in range(n_chunks)],
            [pltpu.SemaphoreType.DMA for _ in range(NBUF)],
            [pltpu.SemaphoreType.DMA for _ in range(NBUF)],
        ],
    )
    def _posemb(w_hbm, out_hbm, bufs, idxs, sins, souts):
        wid = lax.axis_index("s") * NC + lax.axis_index("c")
        base = wid * rows_per_w

        lane = lax.iota(jnp.int32, L)
        for ci in range(n_chunks):
            for j in range(CHUNK // L):
                idxs[ci][pl.ds(j * L, L)] = (
                    _POS_OFFSET + base + ci * CHUNK + j * L) + lane

        def start_in(ci):
            return pltpu.async_copy(
                w_hbm.at[idxs[ci]], bufs[ci % NBUF], sins[ci % NBUF])

        def start_outs(ci):
            r0 = base + ci * CHUNK
            return [
                pltpu.async_copy(
                    bufs[ci % NBUF],
                    out_hbm.at[bi, pl.ds(r0, CHUNK)],
                    souts[ci % NBUF])
                for bi in range(b)
            ]

        ins, outs = {}, {}
        for ci in range(min(NBUF, n_chunks)):
            ins[ci] = start_in(ci)
        for ci in range(min(NBUF, n_chunks)):
            ins[ci].wait()
            outs[ci] = start_outs(ci)
        for ci in range(NBUF, n_chunks):
            for cp in outs[ci - NBUF]:
                cp.wait()
            ins[ci] = start_in(ci)
            ins[ci].wait()
            outs[ci] = start_outs(ci)
        for ci in range(max(0, n_chunks - NBUF), n_chunks):
            for cp in outs[ci]:
                cp.wait()

    return _posemb(weights)
